# Initial kernel scaffold; baseline (speedup 1.0000x reference)
import jax
import jax.numpy as jnp
from jax.experimental import pallas as pl


def kernel(user_emb, item_emb, edge_src, edge_dst, edge_w):
    raise NotImplementedError("write your pallas kernel here")


# speedup vs baseline: 7.0955x; 7.0955x over previous
import functools

import jax
import jax.numpy as jnp
from jax import lax
from jax.experimental import pallas as pl
from jax.experimental.pallas import tpu as pltpu
from jax.experimental.pallas import tpu_sc as plsc

N_USER = 25000
N_ITEM = 25000
N = N_USER + N_ITEM
NE = 800000
H = 32
NT = 16
NPAD = 51200
RPT = NPAD // NT
PT = 50176
EPAD = NT * PT
SB = 1024
CH = 128
NCH = SB // CH
NSB = PT // SB
INV3 = 1.0 / 3.0


def _zero_rows(ref, n):
    def body(i, _):
        z = jnp.zeros((16,), jnp.float32)
        ref[i, pl.ds(0, 16)] = z
        ref[i, pl.ds(16, 16)] = z
        return 0
    lax.fori_loop(0, n, body, 0)


def _body(x_in, src2, dst2, w2, out, xa, xb,
          acc, srcb, dstb, wb, gidx, rows, tmp, t1, t2, gsem):
    c = lax.axis_index("c")
    s = lax.axis_index("s")
    cbase = c * NPAD
    r0 = s * RPT
    e0 = (s * PT) // CH

    for layer in range(3):
        x_src = (x_in, xa, xb)[layer]
        x_dst = (xa, xb, None)[layer]

        plsc.subcore_barrier()
        _zero_rows(tmp, CH)

        def zacc(t, _):
            pltpu.sync_copy(tmp, acc.at[pl.ds(r0 + t * CH, CH)])
            return 0
        lax.fori_loop(0, RPT // CH, zacc, 0)
        plsc.subcore_barrier()

        def superchunk(sc_i, _):
            er = pl.multiple_of(e0 + sc_i * NCH, NCH)
            pltpu.sync_copy(src2.at[pl.ds(er, NCH)], srcb)
            pltpu.sync_copy(dst2.at[pl.ds(er, NCH)], dstb)
            pltpu.sync_copy(w2.at[pl.ds(er, NCH)], wb)

            def gi(i, _):
                k = i // (CH // 16)
                j = (i % (CH // 16)) * 16
                gidx[k, pl.ds(j, 16)] = srcb[k, pl.ds(j, 16)] + cbase
                return 0
            lax.fori_loop(0, NCH * (CH // 16), gi, 0)

            d = pltpu.async_copy(x_src.at[gidx.at[0]], rows.at[0], gsem)
            for k in range(NCH):
                d.wait()
                if k + 1 < NCH:
                    d = pltpu.async_copy(
                        x_src.at[gidx.at[k + 1]], rows.at[(k + 1) % 2], gsem)
                rb = rows.at[k % 2]

                def scale(j, _):
                    wv = wb[k, pl.ds(j * 16, 16)]
                    for l in range(16):
                        b = j * 16 + l
                        ws = wv[l]
                        rb[b, pl.ds(0, 16)] = rb[b, pl.ds(0, 16)] * ws
                        rb[b, pl.ds(16, 16)] = rb[b, pl.ds(16, 16)] * ws
                    return 0
                lax.fori_loop(0, CH // 16, scale, 0)
                pltpu.sync_copy(rb, acc.at[dstb.at[k]], add=True)
            return 0
        lax.fori_loop(0, NSB, superchunk, 0)
        plsc.subcore_barrier()

        if layer < 2:
            pltpu.sync_copy(acc.at[pl.ds(r0, RPT)],
                            x_dst.at[pl.ds(cbase + r0, RPT)])
        else:
            def drain3(t, _):
                rr = r0 + t * CH
                pltpu.sync_copy(acc.at[pl.ds(rr, CH)], tmp)
                pltpu.sync_copy(xa.at[pl.ds(cbase + rr, CH)], t1)
                pltpu.sync_copy(xb.at[pl.ds(cbase + rr, CH)], t2)

                def mean3(b, _):
                    tmp[b, pl.ds(0, 16)] = (
                        t1[b, pl.ds(0, 16)] + t2[b, pl.ds(0, 16)]
                        + tmp[b, pl.ds(0, 16)]) * INV3
                    tmp[b, pl.ds(16, 16)] = (
                        t1[b, pl.ds(16, 16)] + t2[b, pl.ds(16, 16)]
                        + tmp[b, pl.ds(16, 16)]) * INV3
                    return 0
                lax.fori_loop(0, CH, mean3, 0, unroll=4)
                pltpu.sync_copy(tmp, out.at[pl.ds(cbase + rr, CH)])
                return 0
            lax.fori_loop(0, RPT // CH, drain3, 0)


_spmm3 = functools.partial(
    pl.kernel,
    out_type=(
        jax.ShapeDtypeStruct((2 * NPAD, H), jnp.float32),
        jax.ShapeDtypeStruct((2 * NPAD, H), jnp.float32),
        jax.ShapeDtypeStruct((2 * NPAD, H), jnp.float32),
    ),
    mesh=plsc.VectorSubcoreMesh(core_axis_name="c", subcore_axis_name="s",
                                num_cores=2, num_subcores=NT),
    compiler_params=pltpu.CompilerParams(use_tc_tiling_on_sc=False),
    scratch_types=(
        pltpu.VMEM_SHARED((NPAD, H), jnp.float32),
        pltpu.VMEM((NCH, CH), jnp.int32),
        pltpu.VMEM((NCH, CH), jnp.int32),
        pltpu.VMEM((NCH, CH), jnp.float32),
        pltpu.VMEM((NCH, CH), jnp.int32),
        pltpu.VMEM((2, CH, H), jnp.float32),
        pltpu.VMEM((CH, H), jnp.float32),
        pltpu.VMEM((CH, H), jnp.float32),
        pltpu.VMEM((CH, H), jnp.float32),
        pltpu.SemaphoreType.DMA,
    ),
)(_body)


def kernel(user_emb, item_emb, edge_src, edge_dst, edge_w):
    ego = jnp.concatenate([user_emb, item_emb], axis=0)
    x0 = jnp.pad(ego[:, :H], ((0, NPAD - N), (0, 0)))
    x1 = jnp.pad(ego[:, H:], ((0, NPAD - N), (0, 0)))
    x_in = jnp.concatenate([x0, x1], axis=0)
    src = jnp.pad(edge_src.astype(jnp.int32), (0, EPAD - NE))
    dst = jnp.pad(edge_dst.astype(jnp.int32), (0, EPAD - NE))
    w = jnp.pad(edge_w.astype(jnp.float32), (0, EPAD - NE))
    out, _, _ = _spmm3(x_in,
                       src.reshape(EPAD // CH, CH),
                       dst.reshape(EPAD // CH, CH),
                       w.reshape(EPAD // CH, CH))
    final = jnp.concatenate([out[:N], out[NPAD:NPAD + N]], axis=1)
    return final[:N_USER], final[N_USER:]


# speedup vs baseline: 8.1709x; 1.1516x over previous
import functools

import jax
import jax.numpy as jnp
from jax import lax
from jax.experimental import pallas as pl
from jax.experimental.pallas import tpu as pltpu
from jax.experimental.pallas import tpu_sc as plsc

N_USER = 25000
N_ITEM = 25000
N = N_USER + N_ITEM
NE = 800000
H = 32
NT = 16
NPAD = 51200
RPT = NPAD // NT
PT = 50176
EPAD = NT * PT
SB = 1024
CH = 128
NCH = SB // CH
NSB = PT // SB
INV3 = 1.0 / 3.0


def _zero_rows(ref, n):
    def body(i, _):
        z = jnp.zeros((16,), jnp.float32)
        ref[i, pl.ds(0, 16)] = z
        ref[i, pl.ds(16, 16)] = z
        return 0
    lax.fori_loop(0, n, body, 0)


def _body(x_in, src2, dst2, w2, out, xa, xb,
          acc, srcb, dstb, wb, gidx, rows, tmp, esem, gsem, ssem):
    c = lax.axis_index("c")
    s = lax.axis_index("s")
    cbase = c * NPAD
    r0 = s * RPT
    e0 = pl.multiple_of((s * PT) // CH, 8)

    for layer in range(3):
        x_src = (x_in, xa, xb)[layer]
        x_dst = (xa, xb, None)[layer]

        plsc.subcore_barrier()
        _zero_rows(tmp, CH)

        def zacc(t, _):
            pltpu.sync_copy(tmp, acc.at[pl.ds(r0 + t * CH, CH)])
            return 0
        lax.fori_loop(0, RPT // CH, zacc, 0)
        plsc.subcore_barrier()

        pltpu.async_copy(src2.at[pl.ds(e0, NCH)], srcb.at[0], esem)
        pltpu.async_copy(dst2.at[pl.ds(e0, NCH)], dstb.at[0], esem)
        pltpu.async_copy(w2.at[pl.ds(e0, NCH)], wb.at[0], esem)

        def superchunk(sc_i, _):
            bi = sc_i % 2
            er = pl.multiple_of(e0 + sc_i * NCH, NCH)
            pltpu.make_async_copy(
                src2.at[pl.ds(er, NCH)], srcb.at[bi], esem).wait()
            pltpu.make_async_copy(
                dst2.at[pl.ds(er, NCH)], dstb.at[bi], esem).wait()
            pltpu.make_async_copy(
                w2.at[pl.ds(er, NCH)], wb.at[bi], esem).wait()

            @pl.when(sc_i + 1 < NSB)
            def _():
                nb = 1 - bi
                er2 = pl.multiple_of(e0 + (sc_i + 1) * NCH, NCH)
                pltpu.async_copy(src2.at[pl.ds(er2, NCH)], srcb.at[nb], esem)
                pltpu.async_copy(dst2.at[pl.ds(er2, NCH)], dstb.at[nb], esem)
                pltpu.async_copy(w2.at[pl.ds(er2, NCH)], wb.at[nb], esem)

            def gi(i, _):
                k = i // (CH // 16)
                j = (i % (CH // 16)) * 16
                gidx[k, pl.ds(j, 16)] = srcb[bi, k, pl.ds(j, 16)] + cbase
                return 0
            lax.fori_loop(0, NCH * (CH // 16), gi, 0)

            d = pltpu.async_copy(x_src.at[gidx.at[0]], rows.at[0], gsem)
            for k in range(NCH):
                d.wait()
                if k + 1 < NCH:
                    d = pltpu.async_copy(
                        x_src.at[gidx.at[k + 1]], rows.at[(k + 1) % 4], gsem)
                rb = rows.at[k % 4]

                def scale(j, _):
                    wv = wb[bi, k, pl.ds(j * 16, 16)]
                    for l in range(16):
                        b = j * 16 + l
                        ws = wv[l]
                        rb[b, pl.ds(0, 16)] = rb[b, pl.ds(0, 16)] * ws
                        rb[b, pl.ds(16, 16)] = rb[b, pl.ds(16, 16)] * ws
                    return 0
                lax.fori_loop(0, CH // 16, scale, 0)
                pltpu.sync_copy(rb, acc.at[dstb.at[bi, k]], add=True)
            return 0
        lax.fori_loop(0, NSB, superchunk, 0)
        plsc.subcore_barrier()

        if layer < 2:
            pltpu.sync_copy(acc.at[pl.ds(r0, RPT)],
                            x_dst.at[pl.ds(cbase + r0, RPT)])
        else:
            def drain3(t, _):
                rr = r0 + t * CH
                t1 = rows.at[2]
                t2 = rows.at[3]
                pltpu.sync_copy(acc.at[pl.ds(rr, CH)], tmp)
                pltpu.sync_copy(xa.at[pl.ds(cbase + rr, CH)], t1)
                pltpu.sync_copy(xb.at[pl.ds(cbase + rr, CH)], t2)

                def mean3(b, _):
                    tmp[b, pl.ds(0, 16)] = (
                        t1[b, pl.ds(0, 16)] + t2[b, pl.ds(0, 16)]
                        + tmp[b, pl.ds(0, 16)]) * INV3
                    tmp[b, pl.ds(16, 16)] = (
                        t1[b, pl.ds(16, 16)] + t2[b, pl.ds(16, 16)]
                        + tmp[b, pl.ds(16, 16)]) * INV3
                    return 0
                lax.fori_loop(0, CH, mean3, 0, unroll=4)
                pltpu.sync_copy(tmp, out.at[pl.ds(cbase + rr, CH)])
                return 0
            lax.fori_loop(0, RPT // CH, drain3, 0)


_spmm3 = functools.partial(
    pl.kernel,
    out_type=(
        jax.ShapeDtypeStruct((2 * NPAD, H), jnp.float32),
        jax.ShapeDtypeStruct((2 * NPAD, H), jnp.float32),
        jax.ShapeDtypeStruct((2 * NPAD, H), jnp.float32),
    ),
    mesh=plsc.VectorSubcoreMesh(core_axis_name="c", subcore_axis_name="s",
                                num_cores=2, num_subcores=NT),
    compiler_params=pltpu.CompilerParams(use_tc_tiling_on_sc=False),
    scratch_types=(
        pltpu.VMEM_SHARED((NPAD, H), jnp.float32),
        pltpu.VMEM((2, NCH, CH), jnp.int32),
        pltpu.VMEM((2, NCH, CH), jnp.int32),
        pltpu.VMEM((2, NCH, CH), jnp.float32),
        pltpu.VMEM((NCH, CH), jnp.int32),
        pltpu.VMEM((4, CH, H), jnp.float32),
        pltpu.VMEM((CH, H), jnp.float32),
        pltpu.SemaphoreType.DMA,
        pltpu.SemaphoreType.DMA,
        pltpu.SemaphoreType.DMA,
    ),
)(_body)


def kernel(user_emb, item_emb, edge_src, edge_dst, edge_w):
    ego = jnp.concatenate([user_emb, item_emb], axis=0)
    x0 = jnp.pad(ego[:, :H], ((0, NPAD - N), (0, 0)))
    x1 = jnp.pad(ego[:, H:], ((0, NPAD - N), (0, 0)))
    x_in = jnp.concatenate([x0, x1], axis=0)
    src = jnp.pad(edge_src.astype(jnp.int32), (0, EPAD - NE))
    dst = jnp.pad(edge_dst.astype(jnp.int32), (0, EPAD - NE))
    w = jnp.pad(edge_w.astype(jnp.float32), (0, EPAD - NE))
    out, _, _ = _spmm3(x_in,
                       src.reshape(EPAD // CH, CH),
                       dst.reshape(EPAD // CH, CH),
                       w.reshape(EPAD // CH, CH))
    final = jnp.concatenate([out[:N], out[NPAD:NPAD + N]], axis=1)
    return final[:N_USER], final[N_USER:]


# speedup vs baseline: 10.6955x; 1.3090x over previous
import functools

import jax
import jax.numpy as jnp
from jax import lax
from jax.experimental import pallas as pl
from jax.experimental.pallas import tpu as pltpu
from jax.experimental.pallas import tpu_sc as plsc

N_USER = 25000
N_ITEM = 25000
N = N_USER + N_ITEM
NE = 800000
H = 32
NT = 16
NPAD = 51200
RPT = NPAD // NT
PT = 50176
EPAD = NT * PT
SB = 1024
CH = 128
NCH = SB // CH
NSB = PT // SB
INV3 = 1.0 / 3.0


def _zero_rows(ref, n):
    def body(i, _):
        z = jnp.zeros((16,), jnp.float32)
        ref[i, pl.ds(0, 16)] = z
        ref[i, pl.ds(16, 16)] = z
        return 0
    lax.fori_loop(0, n, body, 0)


def _body(x_in, src2, dst2, w2, out, xa, xb,
          acc, srcb, dstb, wb, gidx, rows, tmp, esem, gsem, ssem):
    c = lax.axis_index("c")
    s = lax.axis_index("s")
    cbase = c * NPAD
    r0 = s * RPT
    e0 = pl.multiple_of((s * PT) // CH, 8)

    for layer in range(3):
        x_src = (x_in, xa, xb)[layer]
        x_dst = (xa, xb, None)[layer]

        plsc.subcore_barrier()
        _zero_rows(tmp, CH)

        def zacc(t, _):
            pltpu.sync_copy(tmp, acc.at[pl.ds(r0 + t * CH, CH)])
            return 0
        lax.fori_loop(0, RPT // CH, zacc, 0)
        plsc.subcore_barrier()

        pltpu.async_copy(src2.at[pl.ds(e0, NCH)], srcb.at[0], esem)
        pltpu.async_copy(dst2.at[pl.ds(e0, NCH)], dstb.at[0], esem)
        pltpu.async_copy(w2.at[pl.ds(e0, NCH)], wb.at[0], esem)

        def superchunk(sc_i, _):
            bi = sc_i % 2
            er = pl.multiple_of(e0 + sc_i * NCH, NCH)
            pltpu.make_async_copy(
                src2.at[pl.ds(er, NCH)], srcb.at[bi], esem).wait()
            pltpu.make_async_copy(
                dst2.at[pl.ds(er, NCH)], dstb.at[bi], esem).wait()
            pltpu.make_async_copy(
                w2.at[pl.ds(er, NCH)], wb.at[bi], esem).wait()

            @pl.when(sc_i + 1 < NSB)
            def _():
                nb = 1 - bi
                er2 = pl.multiple_of(e0 + (sc_i + 1) * NCH, NCH)
                pltpu.async_copy(src2.at[pl.ds(er2, NCH)], srcb.at[nb], esem)
                pltpu.async_copy(dst2.at[pl.ds(er2, NCH)], dstb.at[nb], esem)
                pltpu.async_copy(w2.at[pl.ds(er2, NCH)], wb.at[nb], esem)

            def gi(i, _):
                k = i // (CH // 16)
                j = (i % (CH // 16)) * 16
                gidx[k, pl.ds(j, 16)] = srcb[bi, k, pl.ds(j, 16)] + cbase
                return 0
            lax.fori_loop(0, NCH * (CH // 16), gi, 0)

            gd = [None] * NCH
            gd[0] = pltpu.async_copy(x_src.at[gidx.at[0]], rows.at[0], gsem)
            gd[1] = pltpu.async_copy(x_src.at[gidx.at[1]], rows.at[1], gsem)
            for k in range(NCH):
                if k + 2 < NCH:
                    gd[k + 2] = pltpu.async_copy(
                        x_src.at[gidx.at[k + 2]], rows.at[(k + 2) % 4], gsem)
                gd[k].wait()
                rb = rows.at[k % 4]

                def scale(j, _):
                    wv = wb[bi, k, pl.ds(j * 16, 16)]
                    for l in range(16):
                        b = j * 16 + l
                        ws = wv[l]
                        rb[b, pl.ds(0, 16)] = rb[b, pl.ds(0, 16)] * ws
                        rb[b, pl.ds(16, 16)] = rb[b, pl.ds(16, 16)] * ws
                    return 0
                lax.fori_loop(0, CH // 16, scale, 0)
                pltpu.sync_copy(rb, acc.at[dstb.at[bi, k]], add=True)
            return 0
        lax.fori_loop(0, NSB, superchunk, 0)
        plsc.subcore_barrier()

        if layer < 2:
            pltpu.sync_copy(acc.at[pl.ds(r0, RPT)],
                            x_dst.at[pl.ds(cbase + r0, RPT)])
        else:
            def drain3(t, _):
                rr = r0 + t * CH
                t1 = rows.at[2]
                t2 = rows.at[3]
                pltpu.sync_copy(acc.at[pl.ds(rr, CH)], tmp)
                pltpu.sync_copy(xa.at[pl.ds(cbase + rr, CH)], t1)
                pltpu.sync_copy(xb.at[pl.ds(cbase + rr, CH)], t2)

                def mean3(b, _):
                    tmp[b, pl.ds(0, 16)] = (
                        t1[b, pl.ds(0, 16)] + t2[b, pl.ds(0, 16)]
                        + tmp[b, pl.ds(0, 16)]) * INV3
                    tmp[b, pl.ds(16, 16)] = (
                        t1[b, pl.ds(16, 16)] + t2[b, pl.ds(16, 16)]
                        + tmp[b, pl.ds(16, 16)]) * INV3
                    return 0
                lax.fori_loop(0, CH, mean3, 0, unroll=4)
                pltpu.sync_copy(tmp, out.at[pl.ds(cbase + rr, CH)])
                return 0
            lax.fori_loop(0, RPT // CH, drain3, 0)


_spmm3 = functools.partial(
    pl.kernel,
    out_type=(
        jax.ShapeDtypeStruct((2 * NPAD, H), jnp.float32),
        jax.ShapeDtypeStruct((2 * NPAD, H), jnp.float32),
        jax.ShapeDtypeStruct((2 * NPAD, H), jnp.float32),
    ),
    mesh=plsc.VectorSubcoreMesh(core_axis_name="c", subcore_axis_name="s",
                                num_cores=2, num_subcores=NT),
    compiler_params=pltpu.CompilerParams(use_tc_tiling_on_sc=False),
    scratch_types=(
        pltpu.VMEM_SHARED((NPAD, H), jnp.float32),
        pltpu.VMEM((2, NCH, CH), jnp.int32),
        pltpu.VMEM((2, NCH, CH), jnp.int32),
        pltpu.VMEM((2, NCH, CH), jnp.float32),
        pltpu.VMEM((NCH, CH), jnp.int32),
        pltpu.VMEM((4, CH, H), jnp.float32),
        pltpu.VMEM((CH, H), jnp.float32),
        pltpu.SemaphoreType.DMA,
        pltpu.SemaphoreType.DMA,
        pltpu.SemaphoreType.DMA,
    ),
)(_body)


def kernel(user_emb, item_emb, edge_src, edge_dst, edge_w):
    ego = jnp.concatenate([user_emb, item_emb], axis=0)
    x0 = jnp.pad(ego[:, :H], ((0, NPAD - N), (0, 0)))
    x1 = jnp.pad(ego[:, H:], ((0, NPAD - N), (0, 0)))
    x_in = jnp.concatenate([x0, x1], axis=0)
    src = jnp.pad(edge_src.astype(jnp.int32), (0, EPAD - NE))
    dst = jnp.pad(edge_dst.astype(jnp.int32), (0, EPAD - NE))
    w = jnp.pad(edge_w.astype(jnp.float32), (0, EPAD - NE))
    out, _, _ = _spmm3(x_in,
                       src.reshape(EPAD // CH, CH),
                       dst.reshape(EPAD // CH, CH),
                       w.reshape(EPAD // CH, CH))
    final = jnp.concatenate([out[:N], out[NPAD:NPAD + N]], axis=1)
    return final[:N_USER], final[N_USER:]


# speedup vs baseline: 11.6320x; 1.0876x over previous
import functools

import jax
import jax.numpy as jnp
from jax import lax
from jax.experimental import pallas as pl
from jax.experimental.pallas import tpu as pltpu
from jax.experimental.pallas import tpu_sc as plsc

N_USER = 25000
N_ITEM = 25000
N = N_USER + N_ITEM
NE = 800000
H = 32
NT = 16
NPAD = 51200
RPT = NPAD // NT
PT = 50176
EPAD = NT * PT
SB = 1024
CH = 128
NCH = SB // CH
NSB = PT // SB
INV3 = 1.0 / 3.0


def _zero_rows(ref, n):
    def body(i, _):
        z = jnp.zeros((16,), jnp.float32)
        ref[i, pl.ds(0, 16)] = z
        ref[i, pl.ds(16, 16)] = z
        return 0
    lax.fori_loop(0, n, body, 0)


def _body(x_in, src2, dst2, w2, out, xa, xb,
          acc, srcb, dstb, wb, gidx, rows, tmp, esem, gsem, ssem):
    c = lax.axis_index("c")
    s = lax.axis_index("s")
    cbase = c * NPAD
    r0 = s * RPT
    e0 = pl.multiple_of((s * PT) // CH, 8)

    for layer in range(3):
        x_src = (x_in, xa, xb)[layer]
        x_dst = (xa, xb, None)[layer]

        plsc.subcore_barrier()
        _zero_rows(tmp, CH)

        def zacc(t, _):
            pltpu.sync_copy(tmp, acc.at[pl.ds(r0 + t * CH, CH)])
            return 0
        lax.fori_loop(0, RPT // CH, zacc, 0)
        plsc.subcore_barrier()

        pltpu.async_copy(src2.at[pl.ds(e0, NCH)], srcb.at[0], esem)
        pltpu.async_copy(dst2.at[pl.ds(e0, NCH)], dstb.at[0], esem)
        pltpu.async_copy(w2.at[pl.ds(e0, NCH)], wb.at[0], esem)

        def superchunk(sc_i, _):
            bi = sc_i % 2
            er = pl.multiple_of(e0 + sc_i * NCH, NCH)
            pltpu.make_async_copy(
                src2.at[pl.ds(er, NCH)], srcb.at[bi], esem).wait()
            pltpu.make_async_copy(
                dst2.at[pl.ds(er, NCH)], dstb.at[bi], esem).wait()
            pltpu.make_async_copy(
                w2.at[pl.ds(er, NCH)], wb.at[bi], esem).wait()

            @pl.when(sc_i + 1 < NSB)
            def _():
                nb = 1 - bi
                er2 = pl.multiple_of(e0 + (sc_i + 1) * NCH, NCH)
                pltpu.async_copy(src2.at[pl.ds(er2, NCH)], srcb.at[nb], esem)
                pltpu.async_copy(dst2.at[pl.ds(er2, NCH)], dstb.at[nb], esem)
                pltpu.async_copy(w2.at[pl.ds(er2, NCH)], wb.at[nb], esem)

            def gi(i, _):
                k = i // (CH // 16)
                j = (i % (CH // 16)) * 16
                gidx[k, pl.ds(j, 16)] = srcb[bi, k, pl.ds(j, 16)] + cbase
                return 0
            lax.fori_loop(0, NCH * (CH // 16), gi, 0)

            gd = [None] * NCH
            sd = [None] * NCH
            gd[0] = pltpu.async_copy(x_src.at[gidx.at[0]], rows.at[0], gsem)
            gd[1] = pltpu.async_copy(x_src.at[gidx.at[1]], rows.at[1], gsem)
            for k in range(NCH):
                if k >= 2:
                    sd[k - 2].wait()
                if k + 2 < NCH:
                    gd[k + 2] = pltpu.async_copy(
                        x_src.at[gidx.at[k + 2]], rows.at[(k + 2) % 4], gsem)
                gd[k].wait()
                rb = rows.at[k % 4]

                def scale(j, _):
                    wv = wb[bi, k, pl.ds(j * 16, 16)]
                    for l in range(16):
                        b = j * 16 + l
                        ws = wv[l]
                        rb[b, pl.ds(0, 16)] = rb[b, pl.ds(0, 16)] * ws
                        rb[b, pl.ds(16, 16)] = rb[b, pl.ds(16, 16)] * ws
                    return 0
                lax.fori_loop(0, CH // 16, scale, 0)
                sd[k] = pltpu.async_copy(
                    rb, acc.at[dstb.at[bi, k]], ssem, add=True)
            sd[NCH - 2].wait()
            sd[NCH - 1].wait()
            return 0
        lax.fori_loop(0, NSB, superchunk, 0)
        plsc.subcore_barrier()

        if layer < 2:
            pltpu.sync_copy(acc.at[pl.ds(r0, RPT)],
                            x_dst.at[pl.ds(cbase + r0, RPT)])
        else:
            def drain3(t, _):
                rr = r0 + t * CH
                t1 = rows.at[2]
                t2 = rows.at[3]
                pltpu.sync_copy(acc.at[pl.ds(rr, CH)], tmp)
                pltpu.sync_copy(xa.at[pl.ds(cbase + rr, CH)], t1)
                pltpu.sync_copy(xb.at[pl.ds(cbase + rr, CH)], t2)

                def mean3(b, _):
                    tmp[b, pl.ds(0, 16)] = (
                        t1[b, pl.ds(0, 16)] + t2[b, pl.ds(0, 16)]
                        + tmp[b, pl.ds(0, 16)]) * INV3
                    tmp[b, pl.ds(16, 16)] = (
                        t1[b, pl.ds(16, 16)] + t2[b, pl.ds(16, 16)]
                        + tmp[b, pl.ds(16, 16)]) * INV3
                    return 0
                lax.fori_loop(0, CH, mean3, 0, unroll=4)
                pltpu.sync_copy(tmp, out.at[pl.ds(cbase + rr, CH)])
                return 0
            lax.fori_loop(0, RPT // CH, drain3, 0)


_spmm3 = functools.partial(
    pl.kernel,
    out_type=(
        jax.ShapeDtypeStruct((2 * NPAD, H), jnp.float32),
        jax.ShapeDtypeStruct((2 * NPAD, H), jnp.float32),
        jax.ShapeDtypeStruct((2 * NPAD, H), jnp.float32),
    ),
    mesh=plsc.VectorSubcoreMesh(core_axis_name="c", subcore_axis_name="s",
                                num_cores=2, num_subcores=NT),
    compiler_params=pltpu.CompilerParams(use_tc_tiling_on_sc=False),
    scratch_types=(
        pltpu.VMEM_SHARED((NPAD, H), jnp.float32),
        pltpu.VMEM((2, NCH, CH), jnp.int32),
        pltpu.VMEM((2, NCH, CH), jnp.int32),
        pltpu.VMEM((2, NCH, CH), jnp.float32),
        pltpu.VMEM((NCH, CH), jnp.int32),
        pltpu.VMEM((4, CH, H), jnp.float32),
        pltpu.VMEM((CH, H), jnp.float32),
        pltpu.SemaphoreType.DMA,
        pltpu.SemaphoreType.DMA,
        pltpu.SemaphoreType.DMA,
    ),
)(_body)


def kernel(user_emb, item_emb, edge_src, edge_dst, edge_w):
    ego = jnp.concatenate([user_emb, item_emb], axis=0)
    x0 = jnp.pad(ego[:, :H], ((0, NPAD - N), (0, 0)))
    x1 = jnp.pad(ego[:, H:], ((0, NPAD - N), (0, 0)))
    x_in = jnp.concatenate([x0, x1], axis=0)
    src = jnp.pad(edge_src.astype(jnp.int32), (0, EPAD - NE))
    dst = jnp.pad(edge_dst.astype(jnp.int32), (0, EPAD - NE))
    w = jnp.pad(edge_w.astype(jnp.float32), (0, EPAD - NE))
    out, _, _ = _spmm3(x_in,
                       src.reshape(EPAD // CH, CH),
                       dst.reshape(EPAD // CH, CH),
                       w.reshape(EPAD // CH, CH))
    final = jnp.concatenate([out[:N], out[NPAD:NPAD + N]], axis=1)
    return final[:N_USER], final[N_USER:]


# speedup vs baseline: 11.6813x; 1.0042x over previous
import functools

import jax
import jax.numpy as jnp
from jax import lax
from jax.experimental import pallas as pl
from jax.experimental.pallas import tpu as pltpu
from jax.experimental.pallas import tpu_sc as plsc

N_USER = 25000
N_ITEM = 25000
N = N_USER + N_ITEM
NE = 800000
H = 32
NT = 16
NPAD = 51200
RPT = NPAD // NT
PT = 50176
EPAD = NT * PT
SB = 1024
CH = 128
NCH = SB // CH
NSB = PT // SB
INV3 = 1.0 / 3.0


def _zero_rows(ref, n):
    def body(i, _):
        z = jnp.zeros((16,), jnp.float32)
        ref[i, pl.ds(0, 16)] = z
        ref[i, pl.ds(16, 16)] = z
        return 0
    lax.fori_loop(0, n, body, 0)


def _body(x_in, src2, dst2, w2, out, xa, xb,
          acc, srcb, dstb, wb, gidx, rows, tmp, esem, gsem, ssem):
    c = lax.axis_index("c")
    s = lax.axis_index("s")
    cbase = c * NPAD
    r0 = s * RPT
    e0 = pl.multiple_of((s * PT) // CH, 8)

    for layer in range(3):
        x_src = (x_in, xa, xb)[layer]
        x_dst = (xa, xb, None)[layer]

        plsc.subcore_barrier()
        _zero_rows(tmp, CH)

        for t0 in range(0, RPT // CH, 8):
            zd = [pltpu.async_copy(tmp, acc.at[pl.ds(r0 + t * CH, CH)], ssem)
                  for t in range(t0, min(t0 + 8, RPT // CH))]
            for d in zd:
                d.wait()
        plsc.subcore_barrier()

        pltpu.async_copy(src2.at[pl.ds(e0, NCH)], srcb.at[0], esem)
        pltpu.async_copy(dst2.at[pl.ds(e0, NCH)], dstb.at[0], esem)
        pltpu.async_copy(w2.at[pl.ds(e0, NCH)], wb.at[0], esem)

        def superchunk(sc_i, _):
            bi = sc_i % 2
            er = pl.multiple_of(e0 + sc_i * NCH, NCH)
            pltpu.make_async_copy(
                src2.at[pl.ds(er, NCH)], srcb.at[bi], esem).wait()
            pltpu.make_async_copy(
                dst2.at[pl.ds(er, NCH)], dstb.at[bi], esem).wait()
            pltpu.make_async_copy(
                w2.at[pl.ds(er, NCH)], wb.at[bi], esem).wait()

            @pl.when(sc_i + 1 < NSB)
            def _():
                nb = 1 - bi
                er2 = pl.multiple_of(e0 + (sc_i + 1) * NCH, NCH)
                pltpu.async_copy(src2.at[pl.ds(er2, NCH)], srcb.at[nb], esem)
                pltpu.async_copy(dst2.at[pl.ds(er2, NCH)], dstb.at[nb], esem)
                pltpu.async_copy(w2.at[pl.ds(er2, NCH)], wb.at[nb], esem)

            def gi(i, _):
                k = i // (CH // 16)
                j = (i % (CH // 16)) * 16
                gidx[k, pl.ds(j, 16)] = srcb[bi, k, pl.ds(j, 16)] + cbase
                return 0
            lax.fori_loop(0, NCH * (CH // 16), gi, 0)

            gd = [None] * NCH
            sd = [None] * NCH
            gd[0] = pltpu.async_copy(x_src.at[gidx.at[0]], rows.at[0], gsem)
            gd[1] = pltpu.async_copy(x_src.at[gidx.at[1]], rows.at[1], gsem)
            for k in range(NCH):
                if k >= 2:
                    sd[k - 2].wait()
                if k + 2 < NCH:
                    gd[k + 2] = pltpu.async_copy(
                        x_src.at[gidx.at[k + 2]], rows.at[(k + 2) % 4], gsem)
                gd[k].wait()
                rb = rows.at[k % 4]

                def scale(j, _):
                    wv = wb[bi, k, pl.ds(j * 16, 16)]
                    for l in range(16):
                        b = j * 16 + l
                        ws = wv[l]
                        rb[b, pl.ds(0, 16)] = rb[b, pl.ds(0, 16)] * ws
                        rb[b, pl.ds(16, 16)] = rb[b, pl.ds(16, 16)] * ws
                    return 0
                lax.fori_loop(0, CH // 16, scale, 0)
                sd[k] = pltpu.async_copy(
                    rb, acc.at[dstb.at[bi, k]], ssem, add=True)
            sd[NCH - 2].wait()
            sd[NCH - 1].wait()
            return 0
        lax.fori_loop(0, NSB, superchunk, 0)
        plsc.subcore_barrier()

        if layer < 2:
            pltpu.sync_copy(acc.at[pl.ds(r0, RPT)],
                            x_dst.at[pl.ds(cbase + r0, RPT)])
        else:
            def drain3(t, _):
                rr = r0 + t * CH
                t1 = rows.at[2]
                t2 = rows.at[3]
                pltpu.sync_copy(acc.at[pl.ds(rr, CH)], tmp)
                pltpu.sync_copy(xa.at[pl.ds(cbase + rr, CH)], t1)
                pltpu.sync_copy(xb.at[pl.ds(cbase + rr, CH)], t2)

                def mean3(b, _):
                    tmp[b, pl.ds(0, 16)] = (
                        t1[b, pl.ds(0, 16)] + t2[b, pl.ds(0, 16)]
                        + tmp[b, pl.ds(0, 16)]) * INV3
                    tmp[b, pl.ds(16, 16)] = (
                        t1[b, pl.ds(16, 16)] + t2[b, pl.ds(16, 16)]
                        + tmp[b, pl.ds(16, 16)]) * INV3
                    return 0
                lax.fori_loop(0, CH, mean3, 0, unroll=4)
                pltpu.sync_copy(tmp, out.at[pl.ds(cbase + rr, CH)])
                return 0
            lax.fori_loop(0, RPT // CH, drain3, 0)


_spmm3 = functools.partial(
    pl.kernel,
    out_type=(
        jax.ShapeDtypeStruct((2 * NPAD, H), jnp.float32),
        jax.ShapeDtypeStruct((2 * NPAD, H), jnp.float32),
        jax.ShapeDtypeStruct((2 * NPAD, H), jnp.float32),
    ),
    mesh=plsc.VectorSubcoreMesh(core_axis_name="c", subcore_axis_name="s",
                                num_cores=2, num_subcores=NT),
    compiler_params=pltpu.CompilerParams(use_tc_tiling_on_sc=False),
    scratch_types=(
        pltpu.VMEM_SHARED((NPAD, H), jnp.float32),
        pltpu.VMEM((2, NCH, CH), jnp.int32),
        pltpu.VMEM((2, NCH, CH), jnp.int32),
        pltpu.VMEM((2, NCH, CH), jnp.float32),
        pltpu.VMEM((NCH, CH), jnp.int32),
        pltpu.VMEM((4, CH, H), jnp.float32),
        pltpu.VMEM((CH, H), jnp.float32),
        pltpu.SemaphoreType.DMA,
        pltpu.SemaphoreType.DMA,
        pltpu.SemaphoreType.DMA,
    ),
)(_body)


def kernel(user_emb, item_emb, edge_src, edge_dst, edge_w):
    ego = jnp.concatenate([user_emb, item_emb], axis=0)
    x0 = jnp.pad(ego[:, :H], ((0, NPAD - N), (0, 0)))
    x1 = jnp.pad(ego[:, H:], ((0, NPAD - N), (0, 0)))
    x_in = jnp.concatenate([x0, x1], axis=0)
    src = jnp.pad(edge_src.astype(jnp.int32), (0, EPAD - NE))
    dst = jnp.pad(edge_dst.astype(jnp.int32), (0, EPAD - NE))
    w = jnp.pad(edge_w.astype(jnp.float32), (0, EPAD - NE))
    out, _, _ = _spmm3(x_in,
                       src.reshape(EPAD // CH, CH),
                       dst.reshape(EPAD // CH, CH),
                       w.reshape(EPAD // CH, CH))
    final = jnp.concatenate([out[:N], out[NPAD:NPAD + N]], axis=1)
    return final[:N_USER], final[N_USER:]
